# Initial kernel scaffold; baseline (speedup 1.0000x reference)
import jax
import jax.numpy as jnp
from jax.experimental import pallas as pl


def kernel(query_points, codes, We1, be1, We2, be2, Wc, bc, Wn1, bn1, Wn2, bn2, Wf1, bf1, Wf2, bf2):
    raise NotImplementedError("write your pallas kernel here")


# speedup vs baseline: 15.6010x; 15.6010x over previous
import functools

import jax
import jax.numpy as jnp
from jax import lax
from jax.experimental import pallas as pl
from jax.experimental.pallas import tpu as pltpu
from jax.experimental.pallas import tpu_sc as plsc

GRID_N = 8
NGRID = GRID_N ** 3
KNN = 8
CD = 64
HID = 64
NLAYERS = 3
XPAD = 16
HEND = XPAD + CD
F = 128
BIGF = 3.0e38

SC_CORES = 2
SC_SUBCORES = 16
SC_WORKERS = SC_CORES * SC_SUBCORES

ROWT = 512
PT = 512


def _silu(v):
    return v * jax.nn.sigmoid(v)


def _mm(a, b):
    return jnp.dot(a, b, preferred_element_type=jnp.float32,
                   precision=lax.Precision.HIGHEST)


def _knn_body(coords_ref, ct_ref, nbr_ref, *, T):
    b = pl.program_id(0)
    t = pl.program_id(1)
    R = coords_ref.shape[0]
    cols = lax.broadcasted_iota(jnp.int32, (R, T), 1)
    rows_local = t * R + lax.broadcasted_iota(jnp.int32, (R, T), 0)
    d2 = jnp.zeros((R, T), jnp.float32)
    for d in range(3):
        cr = coords_ref[:, d][:, None]
        cc = ct_ref[0, d, :][None, :]
        diff = cr - cc
        d2 = d2 + diff * diff
    d2 = jnp.where(rows_local == cols, BIGF, d2)
    picks = []
    for _ in range(KNN):
        mval = jnp.min(d2, axis=1, keepdims=True)
        eq = d2 <= mval
        idx = jnp.min(jnp.where(eq, cols, T), axis=1, keepdims=True)
        picks.append(idx)
        d2 = jnp.where(cols == idx, BIGF, d2)
    nbr_ref[...] = jnp.concatenate(picks, axis=1) + b * T


def _knn(coords_pad, ct, B, T):
    ntile = T // ROWT
    return pl.pallas_call(
        functools.partial(_knn_body, T=T),
        grid=(B, ntile),
        in_specs=[
            pl.BlockSpec((ROWT, XPAD), lambda b, t: (b * (T // ROWT) + t, 0)),
            pl.BlockSpec((1, 8, T), lambda b, t: (b, 0, 0)),
        ],
        out_specs=pl.BlockSpec((ROWT, KNN), lambda b, t: (b * (T // ROWT) + t, 0)),
        out_shape=jax.ShapeDtypeStruct((B * T, KNN), jnp.int32),
    )(coords_pad, ct)


def _sc_gather(xh, idx3, E, chunk, nchunk):
    mesh = plsc.VectorSubcoreMesh(
        core_axis_name="c", subcore_axis_name="s",
        num_cores=SC_CORES, num_subcores=SC_SUBCORES)

    @functools.partial(
        pl.kernel,
        out_type=jax.ShapeDtypeStruct((E, F), jnp.float32),
        mesh=mesh,
        compiler_params=pltpu.CompilerParams(use_tc_tiling_on_sc=False),
        scratch_types=[
            pltpu.VMEM((nchunk, chunk), jnp.int32),
            pltpu.VMEM((chunk, F), jnp.float32),
            pltpu.SemaphoreType.DMA,
        ],
    )
    def k(table_hbm, idx_hbm, out_hbm, idx_v, rows_v, sem):
        wid = lax.axis_index("s") * SC_CORES + lax.axis_index("c")
        base = wid * (nchunk * chunk)
        pltpu.sync_copy(idx_hbm.at[wid], idx_v)
        for c in range(nchunk):
            pltpu.async_copy(table_hbm.at[idx_v.at[c]], rows_v, sem).wait()
            pltpu.sync_copy(rows_v, out_hbm.at[pl.ds(base + c * chunk, chunk)])

    return k(xh, idx3)


def _layer_body(g_ref, xh_ref, we1a_ref, we1b_ref, we1d_ref, be1_ref,
                we2_ref, be2_ref, wc_ref, bc_ref, wn1a_ref, wn1b_ref,
                bn1_ref, wn2_ref, bn2_ref, out_ref):
    P = xh_ref.shape[0]
    E = P * KNN
    g = g_ref[...]
    xr = g[:, 0:XPAD]
    hr = g[:, XPAD:HEND]
    xh = xh_ref[...]
    xc = xh[:, 0:XPAD]
    hc = xh[:, XPAD:HEND]

    xcr = jnp.broadcast_to(xc[:, None, :], (P, KNN, XPAD)).reshape(E, XPAD)
    rel = xr - xcr
    dist = jnp.sqrt(jnp.sum(rel * rel, axis=1, keepdims=True))

    hcw = _mm(hc, we1b_ref[...])
    hcwr = jnp.broadcast_to(hcw[:, None, :], (P, KNN, HID)).reshape(E, HID)
    a1 = _silu(_mm(hr, we1a_ref[...]) + hcwr + dist * we1d_ref[...]
               + be1_ref[...])
    m = _silu(_mm(a1, we2_ref[...]) + be2_ref[...])
    coef = jnp.sum(m * wc_ref[...], axis=1, keepdims=True) + bc_ref[0, 0]
    cmsg = coef * (rel / (dist + 1e-8))

    dx = jnp.mean(cmsg.reshape(P, KNN, XPAD), axis=1)
    ma = jnp.mean(m.reshape(P, KNN, HID), axis=1)
    n1 = _silu(_mm(hc, wn1a_ref[...]) + _mm(ma, wn1b_ref[...]) + bn1_ref[...])
    hn = hc + _mm(n1, wn2_ref[...]) + bn2_ref[...]
    out_ref[...] = jnp.concatenate(
        [xc + dx, hn, jnp.zeros((P, F - HEND), jnp.float32)], axis=1)


def _layer(g, xh, we1a, we1b, we1d, be1, we2, be2, wc, bc,
           wn1a, wn1b, bn1, wn2, bn2, N):
    ntile = N // PT
    wspec = lambda shape: pl.BlockSpec(shape, lambda i: tuple(0 for _ in shape))
    return pl.pallas_call(
        _layer_body,
        grid=(ntile,),
        in_specs=[
            pl.BlockSpec((PT * KNN, F), lambda i: (i, 0)),
            pl.BlockSpec((PT, F), lambda i: (i, 0)),
            wspec((HID, HID)), wspec((HID, HID)), wspec((1, HID)),
            wspec((1, HID)), wspec((HID, HID)), wspec((1, HID)),
            wspec((1, HID)), wspec((1, 1)), wspec((HID, HID)),
            wspec((HID, HID)), wspec((1, HID)), wspec((HID, CD)),
            wspec((1, CD)),
        ],
        out_specs=pl.BlockSpec((PT, F), lambda i: (i, 0)),
        out_shape=jax.ShapeDtypeStruct((N, F), jnp.float32),
    )(g, xh, we1a, we1b, we1d, be1, we2, be2, wc, bc,
      wn1a, wn1b, bn1, wn2, bn2)


def _readout_body(xh_ref, wf1_ref, bf1_ref, wf2_ref, bf2_ref, sel_ref,
                  out_ref):
    xh = xh_ref[...]
    x = xh[:, 0:XPAD]
    h = xh[:, XPAD:HEND]
    a = jnp.maximum(_mm(h, wf1_ref[...]) + bf1_ref[...], 0.0)
    pred = _mm(a, wf2_ref[...]) + bf2_ref[...]
    out_ref[...] = _mm(x, sel_ref[...]) - pred


def _readout(xh, wf1, bf1, wf2, bf2, sel, B, T, n_points, odim):
    qtile = 512
    ntile = n_points // qtile
    wspec = lambda shape: pl.BlockSpec(shape, lambda b, t: tuple(0 for _ in shape))
    return pl.pallas_call(
        _readout_body,
        grid=(B, ntile),
        in_specs=[
            pl.BlockSpec((qtile, F), lambda b, t: (b * (T // qtile) + t, 0)),
            wspec((CD, HID)), wspec((1, HID)), wspec((HID, odim)),
            wspec((1, odim)), wspec((XPAD, odim)),
        ],
        out_specs=pl.BlockSpec((qtile, odim), lambda b, t: (b * ntile + t, 0)),
        out_shape=jax.ShapeDtypeStruct((B * n_points, odim), jnp.float32),
    )(xh, wf1, bf1, wf2, bf2, sel)


def kernel(query_points, codes, We1, be1, We2, be2, Wc, bc, Wn1, bn1,
           Wn2, bn2, Wf1, bf1, Wf2, bf2):
    B, n_points, _ = query_points.shape
    T = n_points + NGRID
    N = B * T
    E = N * KNN
    odim = Wf2.shape[1]

    lin = jnp.linspace(-1.0, 1.0, GRID_N)
    gx, gy, gz = jnp.meshgrid(lin, lin, lin, indexing="ij")
    grid_points = jnp.stack([gx, gy, gz], axis=-1).reshape(-1, 3)
    grid_points = grid_points.astype(jnp.float32)

    coords = jnp.concatenate(
        [query_points, jnp.broadcast_to(grid_points[None], (B, NGRID, 3))],
        axis=1)
    coords_flat = coords.reshape(N, 3)
    coords_pad = jnp.pad(coords_flat, ((0, 0), (0, XPAD - 3)))
    ct = jnp.pad(coords.transpose(0, 2, 1), ((0, 0), (0, 5), (0, 0)))

    nbr = _knn(coords_pad, ct, B, T)

    h0 = jnp.concatenate(
        [jnp.zeros((B, n_points, CD), jnp.float32), codes], axis=1)
    xh = jnp.concatenate(
        [coords_pad, h0.reshape(N, CD),
         jnp.zeros((N, F - HEND), jnp.float32)], axis=1)

    per_w = E // SC_WORKERS
    chunk = 576
    nchunk = per_w // chunk
    idx3 = nbr.reshape(SC_WORKERS, nchunk, chunk)

    for l in range(NLAYERS):
        g = _sc_gather(xh, idx3, E, chunk, nchunk)
        xh = _layer(
            g, xh,
            We1[l, 0:HID], We1[l, HID:2 * HID], We1[l, 2 * HID:2 * HID + 1],
            be1[l][None, :],
            We2[l], be2[l][None, :], Wc[l].T, bc[l][None, :],
            Wn1[l, 0:CD], Wn1[l, CD:CD + HID], bn1[l][None, :],
            Wn2[l], bn2[l][None, :], N)

    di = lax.broadcasted_iota(jnp.int32, (XPAD, odim), 0)
    ji = lax.broadcasted_iota(jnp.int32, (XPAD, odim), 1)
    sel = (di == ji % 3).astype(jnp.float32)

    y = _readout(xh, Wf1, bf1[None, :], Wf2, bf2[None, :], sel,
                 B, T, n_points, odim)
    return y.reshape(B, n_points, odim // 3, 3)


# speedup vs baseline: 17.6219x; 1.1295x over previous
import functools

import jax
import jax.numpy as jnp
from jax import lax
from jax.experimental import pallas as pl
from jax.experimental.pallas import tpu as pltpu
from jax.experimental.pallas import tpu_sc as plsc

GRID_N = 8
NGRID = GRID_N ** 3
KNN = 8
CD = 64
HID = 64
NLAYERS = 3
XPAD = 16
HEND = XPAD + CD
F = 128
BIGF = 3.0e38

SC_CORES = 2
SC_SUBCORES = 16
SC_WORKERS = SC_CORES * SC_SUBCORES

ROWT = 512
PT = 512


def _silu(v):
    return v * jax.nn.sigmoid(v)


def _mm(a, b):
    return jnp.dot(a, b, preferred_element_type=jnp.float32,
                   precision=lax.Precision.HIGHEST)


_mmh = _mm


def _knn_body(coords_ref, ct_ref, nbr_ref, *, T):
    b = pl.program_id(0)
    t = pl.program_id(1)
    R = coords_ref.shape[0]
    cols = lax.broadcasted_iota(jnp.int32, (R, T), 1)
    rows_local = t * R + lax.broadcasted_iota(jnp.int32, (R, T), 0)
    cr = coords_ref[...]
    ct = ct_ref[0]
    dot = _mmh(cr, ct)
    sqr = jnp.sum(cr * cr, axis=1, keepdims=True)
    sqc = jnp.sum(ct * ct, axis=0, keepdims=True)
    d2 = (sqr - 2.0 * dot) + sqc
    d2 = jnp.where(rows_local == cols, BIGF, d2)
    picks = []
    for _ in range(KNN):
        mval = jnp.min(d2, axis=1, keepdims=True)
        eq = d2 <= mval
        idx = jnp.min(jnp.where(eq, cols, T), axis=1, keepdims=True)
        picks.append(idx)
        d2 = jnp.where(cols == idx, BIGF, d2)
    nbr_ref[...] = jnp.concatenate(picks, axis=1) + b * T


def _knn(coords_pad, ct, B, T):
    ntile = T // ROWT
    return pl.pallas_call(
        functools.partial(_knn_body, T=T),
        grid=(B, ntile),
        in_specs=[
            pl.BlockSpec((ROWT, 8), lambda b, t: (b * (T // ROWT) + t, 0)),
            pl.BlockSpec((1, 8, T), lambda b, t: (b, 0, 0)),
        ],
        out_specs=pl.BlockSpec((ROWT, KNN), lambda b, t: (b * (T // ROWT) + t, 0)),
        out_shape=jax.ShapeDtypeStruct((B * T, KNN), jnp.int32),
    )(coords_pad, ct)


def _sc_gather(xh, idx3, E, chunk, nchunk):
    mesh = plsc.VectorSubcoreMesh(
        core_axis_name="c", subcore_axis_name="s",
        num_cores=SC_CORES, num_subcores=SC_SUBCORES)

    @functools.partial(
        pl.kernel,
        out_type=jax.ShapeDtypeStruct((E, F), jnp.float32),
        mesh=mesh,
        compiler_params=pltpu.CompilerParams(use_tc_tiling_on_sc=False),
        scratch_types=[
            pltpu.VMEM((nchunk, chunk), jnp.int32),
            pltpu.VMEM((chunk, F), jnp.float32),
            pltpu.SemaphoreType.DMA,
        ],
    )
    def k(table_hbm, idx_hbm, out_hbm, idx_v, rows_v, sem):
        wid = lax.axis_index("s") * SC_CORES + lax.axis_index("c")
        base = wid * (nchunk * chunk)
        pltpu.sync_copy(idx_hbm.at[wid], idx_v)
        for c in range(nchunk):
            pltpu.async_copy(table_hbm.at[idx_v.at[c]], rows_v, sem).wait()
            pltpu.sync_copy(rows_v, out_hbm.at[pl.ds(base + c * chunk, chunk)])

    return k(xh, idx3)


def _layer_body(g_ref, xh_ref, we1a_ref, we1b_ref, we1d_ref, be1_ref,
                we2_ref, be2_ref, wc_ref, bc_ref, wn1a_ref, wn1b_ref,
                bn1_ref, wn2_ref, bn2_ref, out_ref):
    P = xh_ref.shape[0]
    xh = xh_ref[...]
    xc = xh[:, 0:XPAD]
    hc = xh[:, XPAD:HEND]

    hcw = _mm(hc, we1b_ref[...]) + be1_ref[...]
    acc_m = jnp.zeros((P, HID), jnp.float32)
    acc_dx = jnp.zeros((P, XPAD), jnp.float32)
    for k in range(KNN):
        xr = g_ref[k, :, 0:XPAD]
        hr = g_ref[k, :, XPAD:HEND]
        rel = xr - xc
        dist = jnp.sqrt(jnp.sum(rel * rel, axis=1, keepdims=True))
        a1 = _silu(_mm(hr, we1a_ref[...]) + hcw + dist * we1d_ref[...])
        m = _silu(_mm(a1, we2_ref[...]) + be2_ref[...])
        coef = jnp.sum(m * wc_ref[...], axis=1, keepdims=True) + bc_ref[0, 0]
        acc_dx = acc_dx + coef * (rel / (dist + 1e-8))
        acc_m = acc_m + m

    ma = acc_m * (1.0 / KNN)
    dx = acc_dx * (1.0 / KNN)
    n1 = _silu(_mm(hc, wn1a_ref[...]) + _mm(ma, wn1b_ref[...]) + bn1_ref[...])
    hn = hc + _mm(n1, wn2_ref[...]) + bn2_ref[...]
    out_ref[...] = jnp.concatenate(
        [xc + dx, hn, jnp.zeros((P, F - HEND), jnp.float32)], axis=1)


def _layer(g, xh, we1a, we1b, we1d, be1, we2, be2, wc, bc,
           wn1a, wn1b, bn1, wn2, bn2, N):
    ntile = N // PT
    wspec = lambda shape: pl.BlockSpec(shape, lambda i: tuple(0 for _ in shape))
    return pl.pallas_call(
        _layer_body,
        grid=(ntile,),
        in_specs=[
            pl.BlockSpec((KNN, PT, F), lambda i: (0, i, 0)),
            pl.BlockSpec((PT, F), lambda i: (i, 0)),
            wspec((HID, HID)), wspec((HID, HID)), wspec((1, HID)),
            wspec((1, HID)), wspec((HID, HID)), wspec((1, HID)),
            wspec((1, HID)), wspec((1, 1)), wspec((HID, HID)),
            wspec((HID, HID)), wspec((1, HID)), wspec((HID, CD)),
            wspec((1, CD)),
        ],
        out_specs=pl.BlockSpec((PT, F), lambda i: (i, 0)),
        out_shape=jax.ShapeDtypeStruct((N, F), jnp.float32),
    )(g, xh, we1a, we1b, we1d, be1, we2, be2, wc, bc,
      wn1a, wn1b, bn1, wn2, bn2)


def _readout_body(xh_ref, wf1_ref, bf1_ref, wf2_ref, bf2_ref, sel_ref,
                  out_ref):
    xh = xh_ref[...]
    x = xh[:, 0:XPAD]
    h = xh[:, XPAD:HEND]
    a = jnp.maximum(_mm(h, wf1_ref[...]) + bf1_ref[...], 0.0)
    pred = _mm(a, wf2_ref[...]) + bf2_ref[...]
    out_ref[...] = _mm(x, sel_ref[...]) - pred


def _readout(xh, wf1, bf1, wf2, bf2, sel, B, T, n_points, odim):
    qtile = 512
    ntile = n_points // qtile
    wspec = lambda shape: pl.BlockSpec(shape, lambda b, t: tuple(0 for _ in shape))
    return pl.pallas_call(
        _readout_body,
        grid=(B, ntile),
        in_specs=[
            pl.BlockSpec((qtile, F), lambda b, t: (b * (T // qtile) + t, 0)),
            wspec((CD, HID)), wspec((1, HID)), wspec((HID, odim)),
            wspec((1, odim)), wspec((XPAD, odim)),
        ],
        out_specs=pl.BlockSpec((qtile, odim), lambda b, t: (b * ntile + t, 0)),
        out_shape=jax.ShapeDtypeStruct((B * n_points, odim), jnp.float32),
    )(xh, wf1, bf1, wf2, bf2, sel)


def kernel(query_points, codes, We1, be1, We2, be2, Wc, bc, Wn1, bn1,
           Wn2, bn2, Wf1, bf1, Wf2, bf2):
    B, n_points, _ = query_points.shape
    T = n_points + NGRID
    N = B * T
    E = N * KNN
    odim = Wf2.shape[1]

    lin = jnp.linspace(-1.0, 1.0, GRID_N)
    gx, gy, gz = jnp.meshgrid(lin, lin, lin, indexing="ij")
    grid_points = jnp.stack([gx, gy, gz], axis=-1).reshape(-1, 3)
    grid_points = grid_points.astype(jnp.float32)

    coords = jnp.concatenate(
        [query_points, jnp.broadcast_to(grid_points[None], (B, NGRID, 3))],
        axis=1)
    coords_flat = coords.reshape(N, 3)
    coords_pad = jnp.pad(coords_flat, ((0, 0), (0, XPAD - 3)))
    coords8 = jnp.pad(coords_flat, ((0, 0), (0, 5)))
    ct = jnp.pad(coords.transpose(0, 2, 1), ((0, 0), (0, 5), (0, 0)))

    nbr = _knn(coords8, ct, B, T)

    h0 = jnp.concatenate(
        [jnp.zeros((B, n_points, CD), jnp.float32), codes], axis=1)
    xh = jnp.concatenate(
        [coords_pad, h0.reshape(N, CD),
         jnp.zeros((N, F - HEND), jnp.float32)], axis=1)

    per_w = E // SC_WORKERS
    chunk = 576
    nchunk = per_w // chunk
    idx3 = nbr.T.reshape(SC_WORKERS, nchunk, chunk)

    for l in range(NLAYERS):
        g = _sc_gather(xh, idx3, E, chunk, nchunk).reshape(KNN, N, F)
        xh = _layer(
            g, xh,
            We1[l, 0:HID], We1[l, HID:2 * HID], We1[l, 2 * HID:2 * HID + 1],
            be1[l][None, :],
            We2[l], be2[l][None, :], Wc[l].T, bc[l][None, :],
            Wn1[l, 0:CD], Wn1[l, CD:CD + HID], bn1[l][None, :],
            Wn2[l], bn2[l][None, :], N)

    di = lax.broadcasted_iota(jnp.int32, (XPAD, odim), 0)
    ji = lax.broadcasted_iota(jnp.int32, (XPAD, odim), 1)
    sel = (di == ji % 3).astype(jnp.float32)

    y = _readout(xh, Wf1, bf1[None, :], Wf2, bf2[None, :], sel,
                 B, T, n_points, odim)
    return y.reshape(B, n_points, odim // 3, 3)


# speedup vs baseline: 21.9002x; 1.2428x over previous
import functools

import jax
import jax.numpy as jnp
from jax import lax
from jax.experimental import pallas as pl
from jax.experimental.pallas import tpu as pltpu
from jax.experimental.pallas import tpu_sc as plsc

GRID_N = 8
NGRID = GRID_N ** 3
KNN = 8
CD = 64
HID = 64
NLAYERS = 3
XPAD = 16
HEND = XPAD + CD
F = 128
BIGF = 3.0e38

SC_CORES = 2
SC_SUBCORES = 16
SC_WORKERS = SC_CORES * SC_SUBCORES

ROWT = 512
PT = 512


def _silu(v):
    return v * jax.nn.sigmoid(v)


def _mm(a, b):
    return jnp.dot(a, b, preferred_element_type=jnp.float32,
                   precision=lax.Precision.HIGHEST)


_mmh = _mm


def _knn_body(coords_ref, ct_ref, nbr_ref, *, T):
    b = pl.program_id(0)
    t = pl.program_id(1)
    R = coords_ref.shape[0]
    cols = lax.broadcasted_iota(jnp.int32, (R, T), 1)
    rows_local = t * R + lax.broadcasted_iota(jnp.int32, (R, T), 0)
    d2 = jnp.zeros((R, T), jnp.float32)
    for d in range(3):
        diff = coords_ref[:, d][:, None] - ct_ref[0, d, :][None, :]
        d2 = d2 + diff * diff
    key = (lax.bitcast_convert_type(d2, jnp.int32) & ~0x1FFF) | cols
    key = jnp.where(rows_local == cols, jnp.int32(0x7FFFFFFF), key)
    picks = []
    for _ in range(KNN):
        mval = jnp.min(key, axis=1, keepdims=True)
        picks.append(mval & 0x1FFF)
        key = jnp.where(key == mval, jnp.int32(0x7FFFFFFF), key)
    nbr_ref[...] = jnp.concatenate(picks, axis=1) + b * T


def _knn(coords_pad, ct, B, T):
    ntile = T // ROWT
    return pl.pallas_call(
        functools.partial(_knn_body, T=T),
        grid=(B, ntile),
        in_specs=[
            pl.BlockSpec((ROWT, 8), lambda b, t: (b * (T // ROWT) + t, 0)),
            pl.BlockSpec((1, 8, T), lambda b, t: (b, 0, 0)),
        ],
        out_specs=pl.BlockSpec((ROWT, KNN), lambda b, t: (b * (T // ROWT) + t, 0)),
        out_shape=jax.ShapeDtypeStruct((B * T, KNN), jnp.int32),
    )(coords_pad, ct)


def _sc_gather(xh, idx3, E, chunk, nchunk):
    mesh = plsc.VectorSubcoreMesh(
        core_axis_name="c", subcore_axis_name="s",
        num_cores=SC_CORES, num_subcores=SC_SUBCORES)

    @functools.partial(
        pl.kernel,
        out_type=jax.ShapeDtypeStruct((E, F), jnp.float32),
        mesh=mesh,
        compiler_params=pltpu.CompilerParams(use_tc_tiling_on_sc=False),
        scratch_types=[
            pltpu.VMEM((nchunk, chunk), jnp.int32),
            pltpu.VMEM((chunk, F), jnp.float32),
            pltpu.SemaphoreType.DMA,
        ],
    )
    def k(table_hbm, idx_hbm, out_hbm, idx_v, rows_v, sem):
        wid = lax.axis_index("s") * SC_CORES + lax.axis_index("c")
        base = wid * (nchunk * chunk)
        pltpu.sync_copy(idx_hbm.at[wid], idx_v)
        for c in range(nchunk):
            pltpu.async_copy(table_hbm.at[idx_v.at[c]], rows_v, sem).wait()
            pltpu.sync_copy(rows_v, out_hbm.at[pl.ds(base + c * chunk, chunk)])

    return k(xh, idx3)


def _layer_body(g_ref, xh_ref, we1a_ref, we1b_ref, we1d_ref, be1_ref,
                we2_ref, be2_ref, wc_ref, bc_ref, wn1a_ref, wn1b_ref,
                bn1_ref, wn2_ref, bn2_ref, out_ref):
    P = xh_ref.shape[0]
    xh = xh_ref[...]
    xc = xh[:, 0:XPAD]
    hc = xh[:, XPAD:HEND]

    hcw = _mm(hc, we1b_ref[...]) + be1_ref[...]
    acc_m = jnp.zeros((P, HID), jnp.float32)
    acc_dx = jnp.zeros((P, XPAD), jnp.float32)
    for k in range(KNN):
        xr = g_ref[k, :, 0:XPAD]
        hr = g_ref[k, :, XPAD:HEND]
        rel = xr - xc
        dist = jnp.sqrt(jnp.sum(rel * rel, axis=1, keepdims=True))
        a1 = _silu(_mm(hr, we1a_ref[...]) + hcw + dist * we1d_ref[...])
        m = _silu(_mm(a1, we2_ref[...]) + be2_ref[...])
        coef = jnp.sum(m * wc_ref[...], axis=1, keepdims=True) + bc_ref[0, 0]
        acc_dx = acc_dx + coef * (rel / (dist + 1e-8))
        acc_m = acc_m + m

    ma = acc_m * (1.0 / KNN)
    dx = acc_dx * (1.0 / KNN)
    n1 = _silu(_mm(hc, wn1a_ref[...]) + _mm(ma, wn1b_ref[...]) + bn1_ref[...])
    hn = hc + _mm(n1, wn2_ref[...]) + bn2_ref[...]
    out_ref[...] = jnp.concatenate(
        [xc + dx, hn, jnp.zeros((P, F - HEND), jnp.float32)], axis=1)


def _layer(g, xh, we1a, we1b, we1d, be1, we2, be2, wc, bc,
           wn1a, wn1b, bn1, wn2, bn2, N):
    ntile = N // PT
    wspec = lambda shape: pl.BlockSpec(shape, lambda i: tuple(0 for _ in shape))
    return pl.pallas_call(
        _layer_body,
        grid=(ntile,),
        in_specs=[
            pl.BlockSpec((KNN, PT, F), lambda i: (0, i, 0)),
            pl.BlockSpec((PT, F), lambda i: (i, 0)),
            wspec((HID, HID)), wspec((HID, HID)), wspec((1, HID)),
            wspec((1, HID)), wspec((HID, HID)), wspec((1, HID)),
            wspec((1, HID)), wspec((1, 1)), wspec((HID, HID)),
            wspec((HID, HID)), wspec((1, HID)), wspec((HID, CD)),
            wspec((1, CD)),
        ],
        out_specs=pl.BlockSpec((PT, F), lambda i: (i, 0)),
        out_shape=jax.ShapeDtypeStruct((N, F), jnp.float32),
    )(g, xh, we1a, we1b, we1d, be1, we2, be2, wc, bc,
      wn1a, wn1b, bn1, wn2, bn2)


def _readout_body(xh_ref, wf1_ref, bf1_ref, wf2_ref, bf2_ref, sel_ref,
                  out_ref):
    xh = xh_ref[...]
    x = xh[:, 0:XPAD]
    h = xh[:, XPAD:HEND]
    a = jnp.maximum(_mm(h, wf1_ref[...]) + bf1_ref[...], 0.0)
    pred = _mm(a, wf2_ref[...]) + bf2_ref[...]
    out_ref[...] = _mm(x, sel_ref[...]) - pred


def _readout(xh, wf1, bf1, wf2, bf2, sel, B, T, n_points, odim):
    qtile = 512
    ntile = n_points // qtile
    wspec = lambda shape: pl.BlockSpec(shape, lambda b, t: tuple(0 for _ in shape))
    return pl.pallas_call(
        _readout_body,
        grid=(B, ntile),
        in_specs=[
            pl.BlockSpec((qtile, F), lambda b, t: (b * (T // qtile) + t, 0)),
            wspec((CD, HID)), wspec((1, HID)), wspec((HID, odim)),
            wspec((1, odim)), wspec((XPAD, odim)),
        ],
        out_specs=pl.BlockSpec((qtile, odim), lambda b, t: (b * ntile + t, 0)),
        out_shape=jax.ShapeDtypeStruct((B * n_points, odim), jnp.float32),
    )(xh, wf1, bf1, wf2, bf2, sel)


def kernel(query_points, codes, We1, be1, We2, be2, Wc, bc, Wn1, bn1,
           Wn2, bn2, Wf1, bf1, Wf2, bf2):
    B, n_points, _ = query_points.shape
    T = n_points + NGRID
    N = B * T
    E = N * KNN
    odim = Wf2.shape[1]

    lin = jnp.linspace(-1.0, 1.0, GRID_N)
    gx, gy, gz = jnp.meshgrid(lin, lin, lin, indexing="ij")
    grid_points = jnp.stack([gx, gy, gz], axis=-1).reshape(-1, 3)
    grid_points = grid_points.astype(jnp.float32)

    coords = jnp.concatenate(
        [query_points, jnp.broadcast_to(grid_points[None], (B, NGRID, 3))],
        axis=1)
    coords_flat = coords.reshape(N, 3)
    coords_pad = jnp.pad(coords_flat, ((0, 0), (0, XPAD - 3)))
    coords8 = jnp.pad(coords_flat, ((0, 0), (0, 5)))
    ct = jnp.pad(coords.transpose(0, 2, 1), ((0, 0), (0, 5), (0, 0)))

    nbr = _knn(coords8, ct, B, T)

    h0 = jnp.concatenate(
        [jnp.zeros((B, n_points, CD), jnp.float32), codes], axis=1)
    xh = jnp.concatenate(
        [coords_pad, h0.reshape(N, CD),
         jnp.zeros((N, F - HEND), jnp.float32)], axis=1)

    per_w = E // SC_WORKERS
    chunk = 576
    nchunk = per_w // chunk
    idx3 = nbr.T.reshape(SC_WORKERS, nchunk, chunk)

    for l in range(NLAYERS):
        g = _sc_gather(xh, idx3, E, chunk, nchunk).reshape(KNN, N, F)
        xh = _layer(
            g, xh,
            We1[l, 0:HID], We1[l, HID:2 * HID], We1[l, 2 * HID:2 * HID + 1],
            be1[l][None, :],
            We2[l], be2[l][None, :], Wc[l].T, bc[l][None, :],
            Wn1[l, 0:CD], Wn1[l, CD:CD + HID], bn1[l][None, :],
            Wn2[l], bn2[l][None, :], N)

    di = lax.broadcasted_iota(jnp.int32, (XPAD, odim), 0)
    ji = lax.broadcasted_iota(jnp.int32, (XPAD, odim), 1)
    sel = (di == ji % 3).astype(jnp.float32)

    y = _readout(xh, Wf1, bf1[None, :], Wf2, bf2[None, :], sel,
                 B, T, n_points, odim)
    return y.reshape(B, n_points, odim // 3, 3)
